# baseline (speedup 1.0000x reference)
import functools

import jax
import jax.numpy as jnp
from jax import lax
from jax.experimental import pallas as pl
from jax.experimental.pallas import tpu as pltpu
from jax.experimental.pallas import tpu_sc as plsc

G_SEL = 50000
EMBED_DIM = 64
BATCH = 256
ROW_PAD = 128

NC, NS = 2, 16
NW = NC * NS
BK = 2048
K_PAD = 51200
SUB = 100
CH = 800
NSC = CH // SUB
CPW_A = 1
CPW_B = 3

_mesh = plsc.VectorSubcoreMesh(core_axis_name="c", subcore_axis_name="s")


@functools.partial(
    pl.kernel,
    mesh=_mesh,
    out_type=jax.ShapeDtypeStruct((K_PAD, ROW_PAD), jnp.float32),
    scratch_types=[
        pltpu.VMEM((NSC, SUB), jnp.int32),
        pltpu.VMEM((CH, ROW_PAD), jnp.float32),
        pltpu.SemaphoreType.DMA,
    ],
)
def _sc_gather(table_hbm, idx2_hbm, out_hbm, idx_v, rows_v, sem):
    core = lax.axis_index("c")
    sub = lax.axis_index("s")

    def chunk(rbase):
        pltpu.sync_copy(idx2_hbm.at[pl.ds(rbase, NSC)], idx_v)
        copies = [
            pltpu.async_copy(
                table_hbm.at[idx_v.at[j]],
                rows_v.at[pl.ds(j * SUB, SUB)],
                sem,
            )
            for j in range(NSC)
        ]
        for c in copies:
            c.wait()
        pltpu.sync_copy(rows_v, out_hbm.at[pl.ds(rbase * SUB, CH)])

    @pl.when(core == 0)
    def _():
        for h in range(CPW_A):
            chunk((sub * CPW_A + h) * NSC)

    @pl.when(core == 1)
    def _():
        for h in range(CPW_B):
            chunk((NS * CPW_A + sub * CPW_B + h) * NSC)


BKV = 4096
NVB = 244
V_TAIL = NVB * BKV


def _pad_body(in_ref, o_ref):
    t = jnp.transpose(in_ref[...])
    o_ref[...] = jnp.concatenate([t, jnp.zeros_like(t)], axis=1)


def _pad_relayout(table_t, vocab):
    return pl.pallas_call(
        _pad_body,
        grid=(NVB,),
        in_specs=[pl.BlockSpec((EMBED_DIM, BKV), lambda k: (0, k))],
        out_specs=pl.BlockSpec((BKV, ROW_PAD), lambda k: (k, 0)),
        out_shape=jax.ShapeDtypeStruct((vocab, ROW_PAD), jnp.float32),
        compiler_params=pltpu.CompilerParams(
            dimension_semantics=("parallel",),
        ),
    )(table_t)


def _mm_body(g_ref, xt_ref, o_ref):
    k = pl.program_id(0)

    @pl.when(k == 0)
    def _():
        o_ref[...] = jnp.zeros_like(o_ref)

    row = k * BK + lax.broadcasted_iota(jnp.int32, (BK, 1), 0)
    xb = jnp.where(row < G_SEL, xt_ref[...], 0.0)
    gb = g_ref[...][:, :EMBED_DIM]
    o_ref[...] += lax.dot_general(
        gb, xb, (((0,), (0,)), ((), ())),
        preferred_element_type=jnp.float32,
    )


def _tc_matmul(g, xt):
    grid = K_PAD // BK
    return pl.pallas_call(
        _mm_body,
        grid=(grid,),
        in_specs=[
            pl.BlockSpec((BK, ROW_PAD), lambda k: (k, 0)),
            pl.BlockSpec((BK, BATCH), lambda k: (k, 0)),
        ],
        out_specs=pl.BlockSpec((EMBED_DIM, BATCH), lambda k: (0, 0)),
        out_shape=jax.ShapeDtypeStruct((EMBED_DIM, BATCH), jnp.float32),
        compiler_params=pltpu.CompilerParams(
            dimension_semantics=("arbitrary",),
        ),
    )(g, xt)


def kernel(x, gene_idx, gene_embeddings):
    vocab = gene_embeddings.shape[0]
    tbl = _pad_relayout(gene_embeddings.T, vocab)
    tail = jnp.pad(
        gene_embeddings[V_TAIL:, :], ((0, 0), (0, ROW_PAD - EMBED_DIM))
    )
    tbl = lax.dynamic_update_slice(tbl, tail, (V_TAIL, 0))
    idx_pad = jnp.concatenate(
        [gene_idx, jnp.zeros((K_PAD - G_SEL,), jnp.int32)]
    )
    idx2d = idx_pad.reshape(K_PAD // SUB, SUB)
    g = _sc_gather(tbl, idx2d)
    out_t = _tc_matmul(g, x.T)
    return out_t.T


# speedup vs baseline: 1.0166x; 1.0166x over previous
import functools

import jax
import jax.numpy as jnp
from jax import lax
from jax.experimental import pallas as pl
from jax.experimental.pallas import tpu as pltpu
from jax.experimental.pallas import tpu_sc as plsc

G_SEL = 50000
EMBED_DIM = 64
BATCH = 256
ROW_PAD = 128

NC, NS = 2, 16
NW = NC * NS
BK = 2048
K_PAD = 51200
SUB = 100
CH = 800
NSC = CH // SUB
CPW_A = 3
CPW_B = 1

_mesh = plsc.VectorSubcoreMesh(core_axis_name="c", subcore_axis_name="s")


@functools.partial(
    pl.kernel,
    mesh=_mesh,
    out_type=jax.ShapeDtypeStruct((K_PAD, ROW_PAD), jnp.float32),
    scratch_types=[
        pltpu.VMEM((NSC, SUB), jnp.int32),
        pltpu.VMEM((CH, ROW_PAD), jnp.float32),
        pltpu.SemaphoreType.DMA,
    ],
)
def _sc_gather(table_hbm, idx2_hbm, out_hbm, idx_v, rows_v, sem):
    core = lax.axis_index("c")
    sub = lax.axis_index("s")

    def chunk(rbase):
        pltpu.sync_copy(idx2_hbm.at[pl.ds(rbase, NSC)], idx_v)
        copies = [
            pltpu.async_copy(
                table_hbm.at[idx_v.at[j]],
                rows_v.at[pl.ds(j * SUB, SUB)],
                sem,
            )
            for j in range(NSC)
        ]
        for c in copies:
            c.wait()
        pltpu.sync_copy(rows_v, out_hbm.at[pl.ds(rbase * SUB, CH)])

    @pl.when(core == 0)
    def _():
        for h in range(CPW_A):
            chunk((sub * CPW_A + h) * NSC)

    @pl.when(core == 1)
    def _():
        for h in range(CPW_B):
            chunk((NS * CPW_A + sub * CPW_B + h) * NSC)


BKV = 4096
NVB = 244
V_TAIL = NVB * BKV


def _pad_body(in_ref, o_ref):
    t = jnp.transpose(in_ref[...])
    o_ref[...] = jnp.concatenate([t, jnp.zeros_like(t)], axis=1)


def _pad_relayout(table_t, vocab):
    return pl.pallas_call(
        _pad_body,
        grid=(NVB,),
        in_specs=[pl.BlockSpec((EMBED_DIM, BKV), lambda k: (0, k))],
        out_specs=pl.BlockSpec((BKV, ROW_PAD), lambda k: (k, 0)),
        out_shape=jax.ShapeDtypeStruct((vocab, ROW_PAD), jnp.float32),
        compiler_params=pltpu.CompilerParams(
            dimension_semantics=("parallel",),
        ),
    )(table_t)


def _mm_body(g_ref, xt_ref, o_ref):
    k = pl.program_id(0)

    @pl.when(k == 0)
    def _():
        o_ref[...] = jnp.zeros_like(o_ref)

    row = k * BK + lax.broadcasted_iota(jnp.int32, (BK, 1), 0)
    xb = jnp.where(row < G_SEL, xt_ref[...], 0.0)
    gb = g_ref[...][:, :EMBED_DIM]
    o_ref[...] += lax.dot_general(
        gb, xb, (((0,), (0,)), ((), ())),
        preferred_element_type=jnp.float32,
    )


def _tc_matmul(g, xt):
    grid = K_PAD // BK
    return pl.pallas_call(
        _mm_body,
        grid=(grid,),
        in_specs=[
            pl.BlockSpec((BK, ROW_PAD), lambda k: (k, 0)),
            pl.BlockSpec((BK, BATCH), lambda k: (k, 0)),
        ],
        out_specs=pl.BlockSpec((EMBED_DIM, BATCH), lambda k: (0, 0)),
        out_shape=jax.ShapeDtypeStruct((EMBED_DIM, BATCH), jnp.float32),
        compiler_params=pltpu.CompilerParams(
            dimension_semantics=("arbitrary",),
        ),
    )(g, xt)


def kernel(x, gene_idx, gene_embeddings):
    vocab = gene_embeddings.shape[0]
    tbl = _pad_relayout(gene_embeddings.T, vocab)
    tail = jnp.pad(
        gene_embeddings[V_TAIL:, :], ((0, 0), (0, ROW_PAD - EMBED_DIM))
    )
    tbl = lax.dynamic_update_slice(tbl, tail, (V_TAIL, 0))
    idx_pad = jnp.concatenate(
        [gene_idx, jnp.zeros((K_PAD - G_SEL,), jnp.int32)]
    )
    idx2d = idx_pad.reshape(K_PAD // SUB, SUB)
    g = _sc_gather(tbl, idx2d)
    out_t = _tc_matmul(g, x.T)
    return out_t.T


# speedup vs baseline: 1.0170x; 1.0003x over previous
import functools

import jax
import jax.numpy as jnp
from jax import lax
from jax.experimental import pallas as pl
from jax.experimental.pallas import tpu as pltpu
from jax.experimental.pallas import tpu_sc as plsc

G_SEL = 50000
EMBED_DIM = 64
BATCH = 256
ROW_PAD = 128

NC, NS = 2, 16
NW = NC * NS
BK = 2048
K_PAD = 51200
BPW = K_PAD // NW
SUB = 100
NSUB = BPW // SUB
HALF = 2

_mesh = plsc.VectorSubcoreMesh(core_axis_name="c", subcore_axis_name="s")


@functools.partial(
    pl.kernel,
    mesh=_mesh,
    out_type=jax.ShapeDtypeStruct((K_PAD, ROW_PAD), jnp.float32),
    scratch_types=[
        pltpu.VMEM((NSUB, SUB), jnp.int32),
        pltpu.VMEM((BPW // HALF, ROW_PAD), jnp.float32),
        pltpu.SemaphoreType.DMA,
    ],
)
def _sc_gather(table_hbm, idx2_hbm, out_hbm, idx_v, rows_v, sem):
    wid = lax.axis_index("c") * NS + lax.axis_index("s")
    base = wid * BPW
    pltpu.sync_copy(idx2_hbm.at[pl.ds(wid * NSUB, NSUB)], idx_v)
    for h in range(HALF):
        copies = [
            pltpu.async_copy(
                table_hbm.at[idx_v.at[h * (NSUB // HALF) + j]],
                rows_v.at[pl.ds(j * SUB, SUB)],
                sem,
            )
            for j in range(NSUB // HALF)
        ]
        for c in copies:
            c.wait()
        dst = pl.ds(base + h * (BPW // HALF), BPW // HALF)
        pltpu.sync_copy(rows_v, out_hbm.at[dst])


BKV = 4096
NVB = 244
V_TAIL = NVB * BKV


def _pad_body(in_ref, o_ref):
    t = jnp.transpose(in_ref[...])
    o_ref[...] = jnp.concatenate([t, jnp.zeros_like(t)], axis=1)


def _pad_relayout(table_t, vocab):
    return pl.pallas_call(
        _pad_body,
        grid=(NVB,),
        in_specs=[pl.BlockSpec((EMBED_DIM, BKV), lambda k: (0, k))],
        out_specs=pl.BlockSpec((BKV, ROW_PAD), lambda k: (k, 0)),
        out_shape=jax.ShapeDtypeStruct((vocab, ROW_PAD), jnp.float32),
        compiler_params=pltpu.CompilerParams(
            dimension_semantics=("parallel",),
        ),
    )(table_t)


def _mm_body(g_ref, xt_ref, o_ref):
    k = pl.program_id(0)

    @pl.when(k == 0)
    def _():
        o_ref[...] = jnp.zeros_like(o_ref)

    row = k * BK + lax.broadcasted_iota(jnp.int32, (BK, 1), 0)
    xb = jnp.where(row < G_SEL, xt_ref[...], 0.0)
    gb = g_ref[...][:, :EMBED_DIM]
    o_ref[...] += lax.dot_general(
        gb, xb, (((0,), (0,)), ((), ())),
        preferred_element_type=jnp.float32,
    )


def _tc_matmul(g, xt):
    grid = K_PAD // BK
    return pl.pallas_call(
        _mm_body,
        grid=(grid,),
        in_specs=[
            pl.BlockSpec((BK, ROW_PAD), lambda k: (k, 0)),
            pl.BlockSpec((BK, BATCH), lambda k: (k, 0)),
        ],
        out_specs=pl.BlockSpec((EMBED_DIM, BATCH), lambda k: (0, 0)),
        out_shape=jax.ShapeDtypeStruct((EMBED_DIM, BATCH), jnp.float32),
        compiler_params=pltpu.CompilerParams(
            dimension_semantics=("arbitrary",),
        ),
    )(g, xt)


def kernel(x, gene_idx, gene_embeddings):
    vocab = gene_embeddings.shape[0]
    tbl = _pad_relayout(gene_embeddings.T, vocab)
    tail = jnp.pad(
        gene_embeddings[V_TAIL:, :], ((0, 0), (0, ROW_PAD - EMBED_DIM))
    )
    tbl = lax.dynamic_update_slice(tbl, tail, (V_TAIL, 0))
    idx_pad = jnp.concatenate(
        [gene_idx, jnp.zeros((K_PAD - G_SEL,), jnp.int32)]
    )
    idx2d = idx_pad.reshape(K_PAD // SUB, SUB)
    g = _sc_gather(tbl, idx2d)
    out_t = _tc_matmul(g, x.T)
    return out_t.T
